# baseline (speedup 1.0000x reference)
import functools

import jax
import jax.numpy as jnp
from jax import lax
from jax.experimental import pallas as pl
from jax.experimental.pallas import tpu as pltpu
from jax.experimental.pallas import tpu_sc as plsc

NC = 2
NS = 16
NW = NC * NS
LANES = 16
C = 128
KPT = 80
NROWS = 2
NBLOB = 4


def _make_deg_kernel(NPAD):
    mesh = plsc.VectorSubcoreMesh(core_axis_name="c", subcore_axis_name="s",
                                  num_cores=NC, num_subcores=NS)
    rows_per_tile = NPAD // NS

    @functools.partial(
        pl.kernel, mesh=mesh,
        out_type=jax.ShapeDtypeStruct((NC, NPAD), jnp.float32),
        compiler_params=pltpu.CompilerParams(needs_layout_passes=False),
        scratch_types=[
            pltpu.VMEM((KPT, C), jnp.int32),
            pltpu.VMEM((KPT, C), jnp.float32),
            pltpu.VMEM((rows_per_tile,), jnp.float32),
            pltpu.VMEM_SHARED((NPAD,), jnp.float32),
        ],
    )
    def deg_kernel(dst_hbm, w_hbm, out_hbm, didx, w_v, zbuf, deg_sh):
        c = lax.axis_index("c")
        s = lax.axis_index("s")
        wid = c * NS + s

        def zero_body(k, _):
            zbuf[pl.ds(k * LANES, LANES)] = jnp.zeros((LANES,), jnp.float32)
            return 0
        lax.fori_loop(0, rows_per_tile // LANES, zero_body, 0)
        pltpu.sync_copy(zbuf, deg_sh.at[pl.ds(s * rows_per_tile, rows_per_tile)])
        pltpu.sync_copy(dst_hbm.at[pl.ds(wid * KPT, KPT)], didx)
        pltpu.sync_copy(w_hbm.at[pl.ds(wid * KPT, KPT)], w_v)
        plsc.subcore_barrier()

        def chunk_body(k, _):
            pltpu.sync_copy(w_v.at[k], deg_sh.at[didx.at[k]], add=True)
            return 0
        lax.fori_loop(0, KPT, chunk_body, 0)
        plsc.subcore_barrier()

        sl = pl.ds(s * rows_per_tile, rows_per_tile)
        pltpu.sync_copy(deg_sh.at[sl], out_hbm.at[c, sl])

    return deg_kernel


def _make_agg_kernel(NPAD, D):
    mesh = plsc.VectorSubcoreMesh(core_axis_name="c", subcore_axis_name="s",
                                  num_cores=NC, num_subcores=NS)
    rows_per_tile = NPAD // NS
    zchunk = 80

    H = C // 2

    @functools.partial(
        pl.kernel, mesh=mesh,
        out_type=jax.ShapeDtypeStruct((NC, NPAD, D), jnp.float32),
        compiler_params=pltpu.CompilerParams(needs_layout_passes=False),
        scratch_types=[
            [pltpu.VMEM((3, C), jnp.int32) for _ in range(NBLOB)],
            [pltpu.VMEM((C, D), jnp.float32) for _ in range(NROWS)],
            [[pltpu.VMEM((H,), jnp.int32) for _ in range(2)] for _ in range(NROWS)],
            pltpu.VMEM_SHARED((NPAD, D), jnp.float32),
            [pltpu.SemaphoreType.DMA for _ in range(NBLOB)],
            [pltpu.SemaphoreType.DMA for _ in range(NROWS)],
            [[pltpu.SemaphoreType.DMA for _ in range(2)] for _ in range(NROWS)],
        ],
    )
    def agg_kernel(hp_hbm, blob_hbm, out_hbm, blob, rows, didx, acc,
                   bsem, gsem, ssem):
        c = lax.axis_index("c")
        s = lax.axis_index("s")
        wid = c * NS + s
        base = wid * KPT

        def zero_row(i, _):
            for j in range(D // LANES):
                rows[0][i, pl.ds(j * LANES, LANES)] = jnp.zeros((LANES,), jnp.float32)
            return 0
        lax.fori_loop(0, C, zero_row, 0)
        for t in range(rows_per_tile // zchunk):
            pltpu.sync_copy(rows[0].at[pl.ds(0, zchunk)],
                            acc.at[pl.ds(s * rows_per_tile + t * zchunk, zchunk)])
        plsc.subcore_barrier()

        def fire_blob(k, bb):
            pltpu.async_copy(blob_hbm.at[base + k], blob[bb], bsem[bb])

        def wait_blob(bb):
            pltpu.make_async_copy(blob_hbm.at[0], blob[bb], bsem[bb]).wait()

        def fire_gather(k, bb, rb):
            pltpu.async_copy(hp_hbm.at[blob[bb].at[0]], rows[rb], gsem[rb])

        def wait_gather(rb):
            pltpu.make_async_copy(hp_hbm.at[pl.ds(0, C)], rows[rb], gsem[rb]).wait()

        def fire_scatter(rb, half):
            sl = pl.ds(half * H, H)
            pltpu.async_copy(rows[rb].at[sl], acc.at[didx[rb][half]],
                             ssem[rb][half], add=True)

        def wait_scatter(rb, half):
            pltpu.make_async_copy(hp_hbm.at[pl.ds(0, H)],
                                  rows[rb].at[pl.ds(0, H)],
                                  ssem[rb][half]).wait()

        fire_blob(0, 0)
        fire_blob(1, 1)
        wait_blob(0)
        fire_gather(0, 0, 0)

        def outer(k4, _):
            for u in range(4):
                k = k4 * 4 + u
                rb = u % NROWS
                rb1 = (u + 1) % NROWS
                bb = u % NBLOB
                bb1 = (u + 1) % NBLOB
                bb2 = (u + 2) % NBLOB

                wait_gather(rb)

                for half in range(2):
                    for j in range(H // LANES):
                        didx[rb][half][pl.ds(j * LANES, LANES)] = (
                            blob[bb][1, pl.ds(half * H + j * LANES, LANES)])

                def scale_row(i, _):
                    wb = plsc.bitcast(
                        plsc.load_gather(
                            blob[bb],
                            [jnp.full((LANES,), 2, jnp.int32),
                             jnp.full((LANES,), 0, jnp.int32) + i]),
                        jnp.float32)
                    for j in range(D // LANES):
                        sl = pl.ds(j * LANES, LANES)
                        rows[rb][i, sl] = rows[rb][i, sl] * wb
                    return 0
                lax.fori_loop(0, H, scale_row, 0, unroll=4)
                fire_scatter(rb, 0)
                lax.fori_loop(H, C, scale_row, 0, unroll=4)
                fire_scatter(rb, 1)

                @pl.when(k + 1 < KPT)
                def _():
                    wait_blob(bb1)

                    @pl.when(k >= 1)
                    def _():
                        wait_scatter(rb1, 0)
                        wait_scatter(rb1, 1)
                    fire_gather(k + 1, bb1, rb1)

                @pl.when(k + 2 < KPT)
                def _():
                    fire_blob(k + 2, bb2)
            return 0
        lax.fori_loop(0, KPT // 4, outer, 0)

        for kk in (KPT - 2, KPT - 1):
            wait_scatter(kk % NROWS, 0)
            wait_scatter(kk % NROWS, 1)
        plsc.subcore_barrier()

        for t in range(rows_per_tile // zchunk):
            sl = pl.ds(s * rows_per_tile + t * zchunk, zchunk)
            pltpu.sync_copy(acc.at[sl], out_hbm.at[c, sl])

    return agg_kernel


_BLK = 1024


def _dis_from_degp(degp_blk):
    deg = degp_blk[0, :] + degp_blk[1, :] + 1.0
    return lax.rsqrt(deg)


def _layer_norm(z, g, b, eps=1e-5):
    mu = jnp.mean(z, axis=-1, keepdims=True)
    var = jnp.mean((z - mu) ** 2, axis=-1, keepdims=True)
    return (z - mu) * lax.rsqrt(var + eps) * g + b


def _tc1_body(degp_ref, x_ref, w_ref, hp_ref):
    dis = _dis_from_degp(degp_ref[...])
    h = jnp.dot(x_ref[...], w_ref[...], preferred_element_type=jnp.float32,
                precision=lax.Precision.HIGHEST)
    hp_ref[...] = h * dis[:, None]


def _tc2_body(p_ref, hp_ref, degp_ref, w2_ref, b1_ref, g1_ref, be1_ref, out_ref):
    dis = _dis_from_degp(degp_ref[...])
    z = (p_ref[0] + p_ref[1] + hp_ref[...]) * dis[:, None] + b1_ref[...]
    z = _layer_norm(z, g1_ref[...], be1_ref[...])
    r = jnp.maximum(z, 0.0)
    h2 = jnp.dot(r, w2_ref[...], preferred_element_type=jnp.float32,
                 precision=lax.Precision.HIGHEST)
    out_ref[...] = h2 * dis[:, None]


def _tc3_body(p_ref, hp_ref, degp_ref, b2_ref, g2_ref, be2_ref, out_ref):
    dis = _dis_from_degp(degp_ref[...])
    z = (p_ref[0] + p_ref[1] + hp_ref[...]) * dis[:, None] + b2_ref[...]
    out_ref[...] = _layer_norm(z, g2_ref[...], be2_ref[...])


def _row_spec(D):
    return pl.BlockSpec((_BLK, D), lambda i: (i, 0))


def _p_spec(D):
    return pl.BlockSpec((2, _BLK, D), lambda i: (0, i, 0))


def _full_spec(shape):
    nd = len(shape)
    return pl.BlockSpec(shape, lambda i: (0,) * nd)


def kernel(x, edge_index, edge_weight, W1, b1, g1, be1, W2, b2, g2, be2):
    N, D = x.shape
    E = edge_weight.shape[0]
    NPAD = ((N + NW * LANES - 1) // (NW * LANES)) * (NW * LANES)

    E_pad = NW * KPT * C
    pad = E_pad - E
    pad_idx = (jnp.arange(pad, dtype=jnp.int32) % N)
    src = jnp.concatenate([edge_index[0].astype(jnp.int32), pad_idx]).reshape(-1, C)
    dst = jnp.concatenate([edge_index[1].astype(jnp.int32), pad_idx]).reshape(-1, C)
    w = jnp.concatenate([edge_weight.astype(jnp.float32),
                         jnp.zeros((pad,), jnp.float32)]).reshape(-1, C)
    wbits = lax.bitcast_convert_type(w, jnp.int32)
    blob = jnp.stack([src, dst, wbits], axis=1)

    degp = _make_deg_kernel(NPAD)(dst, w)
    deg_t = degp[:, :N]

    grid = (N + _BLK - 1) // _BLK
    degp_spec = pl.BlockSpec((2, _BLK), lambda i: (0, i))

    hp1 = pl.pallas_call(
        _tc1_body,
        grid=(grid,),
        in_specs=[degp_spec, _row_spec(D), _full_spec((D, D))],
        out_specs=_row_spec(D),
        out_shape=jax.ShapeDtypeStruct((N, D), jnp.float32),
    )(deg_t, x, W1)

    agg = _make_agg_kernel(NPAD, D)

    P1 = agg(hp1, blob)

    hp2 = pl.pallas_call(
        _tc2_body,
        grid=(grid,),
        in_specs=[_p_spec(D), _row_spec(D), degp_spec, _full_spec((D, D)),
                  _full_spec((1, D)), _full_spec((1, D)), _full_spec((1, D))],
        out_specs=_row_spec(D),
        out_shape=jax.ShapeDtypeStruct((N, D), jnp.float32),
    )(P1, hp1, deg_t, W2, b1.reshape(1, D), g1.reshape(1, D), be1.reshape(1, D))

    P2 = agg(hp2, blob)

    out = pl.pallas_call(
        _tc3_body,
        grid=(grid,),
        in_specs=[_p_spec(D), _row_spec(D), degp_spec,
                  _full_spec((1, D)), _full_spec((1, D)), _full_spec((1, D))],
        out_specs=_row_spec(D),
        out_shape=jax.ShapeDtypeStruct((N, D), jnp.float32),
    )(P2, hp2, deg_t, b2.reshape(1, D), g2.reshape(1, D), be2.reshape(1, D))

    return out


# speedup vs baseline: 1.3228x; 1.3228x over previous
import functools

import jax
import jax.numpy as jnp
from jax import lax
from jax.experimental import pallas as pl
from jax.experimental.pallas import tpu as pltpu
from jax.experimental.pallas import tpu_sc as plsc

NC = 2
NS = 16
NW = NC * NS
LANES = 16
C = 128
KPT = 80
NROWS = 2
NBLOB = 4


def _make_deg_kernel(NPAD):
    mesh = plsc.VectorSubcoreMesh(core_axis_name="c", subcore_axis_name="s",
                                  num_cores=NC, num_subcores=NS)
    rows_per_tile = NPAD // NS

    @functools.partial(
        pl.kernel, mesh=mesh,
        out_type=jax.ShapeDtypeStruct((NC, NPAD), jnp.float32),
        compiler_params=pltpu.CompilerParams(needs_layout_passes=False),
        scratch_types=[
            pltpu.VMEM((KPT, C), jnp.int32),
            pltpu.VMEM((KPT, C), jnp.float32),
            pltpu.VMEM((rows_per_tile,), jnp.float32),
            pltpu.VMEM_SHARED((NPAD,), jnp.float32),
        ],
    )
    def deg_kernel(dst_hbm, w_hbm, out_hbm, didx, w_v, zbuf, deg_sh):
        c = lax.axis_index("c")
        s = lax.axis_index("s")
        wid = c * NS + s

        def zero_body(k, _):
            zbuf[pl.ds(k * LANES, LANES)] = jnp.zeros((LANES,), jnp.float32)
            return 0
        lax.fori_loop(0, rows_per_tile // LANES, zero_body, 0)
        pltpu.sync_copy(zbuf, deg_sh.at[pl.ds(s * rows_per_tile, rows_per_tile)])
        pltpu.sync_copy(dst_hbm.at[pl.ds(wid * KPT, KPT)], didx)
        pltpu.sync_copy(w_hbm.at[pl.ds(wid * KPT, KPT)], w_v)
        plsc.subcore_barrier()

        def chunk_body(k, _):
            pltpu.sync_copy(w_v.at[k], deg_sh.at[didx.at[k]], add=True)
            return 0
        lax.fori_loop(0, KPT, chunk_body, 0)
        plsc.subcore_barrier()

        sl = pl.ds(s * rows_per_tile, rows_per_tile)
        pltpu.sync_copy(deg_sh.at[sl], out_hbm.at[c, sl])

    return deg_kernel


def _make_agg_kernel(NPAD, D):
    mesh = plsc.VectorSubcoreMesh(core_axis_name="c", subcore_axis_name="s",
                                  num_cores=NC, num_subcores=NS)
    rows_per_tile = NPAD // NS
    zchunk = 80

    H = C // 2

    @functools.partial(
        pl.kernel, mesh=mesh,
        out_type=jax.ShapeDtypeStruct((NC, NPAD, D), jnp.float32),
        compiler_params=pltpu.CompilerParams(needs_layout_passes=False),
        scratch_types=[
            [pltpu.VMEM((3, C), jnp.int32) for _ in range(NBLOB)],
            [pltpu.VMEM((C, D), jnp.float32) for _ in range(NROWS)],
            [[pltpu.VMEM((H,), jnp.int32) for _ in range(2)] for _ in range(NROWS)],
            pltpu.VMEM_SHARED((NPAD, D), jnp.float32),
            [pltpu.SemaphoreType.DMA for _ in range(NBLOB)],
            [pltpu.SemaphoreType.DMA for _ in range(NROWS)],
            [[pltpu.SemaphoreType.DMA for _ in range(2)] for _ in range(NROWS)],
        ],
    )
    def agg_kernel(hp_hbm, blob_hbm, out_hbm, blob, rows, didx, acc,
                   bsem, gsem, ssem):
        c = lax.axis_index("c")
        s = lax.axis_index("s")
        wid = c * NS + s
        base = wid * KPT

        def zero_row(i, _):
            for j in range(D // LANES):
                rows[0][i, pl.ds(j * LANES, LANES)] = jnp.zeros((LANES,), jnp.float32)
            return 0
        lax.fori_loop(0, C, zero_row, 0)
        for t in range(rows_per_tile // zchunk):
            pltpu.sync_copy(rows[0].at[pl.ds(0, zchunk)],
                            acc.at[pl.ds(s * rows_per_tile + t * zchunk, zchunk)])
        plsc.subcore_barrier()

        def fire_blob(k, bb):
            pltpu.async_copy(blob_hbm.at[base + k], blob[bb], bsem[bb])

        def wait_blob(bb):
            pltpu.make_async_copy(blob_hbm.at[0], blob[bb], bsem[bb]).wait()

        def fire_gather(k, bb, rb):
            pltpu.async_copy(hp_hbm.at[blob[bb].at[0]], rows[rb], gsem[rb])

        def wait_gather(rb):
            pltpu.make_async_copy(hp_hbm.at[pl.ds(0, C)], rows[rb], gsem[rb]).wait()

        def fire_scatter(rb, half):
            sl = pl.ds(half * H, H)
            pltpu.async_copy(rows[rb].at[sl], acc.at[didx[rb][half]],
                             ssem[rb][half], add=True)

        def wait_scatter(rb, half):
            pltpu.make_async_copy(hp_hbm.at[pl.ds(0, H)],
                                  rows[rb].at[pl.ds(0, H)],
                                  ssem[rb][half]).wait()

        fire_blob(0, 0)
        fire_blob(1, 1)
        wait_blob(0)
        fire_gather(0, 0, 0)

        def outer(k4, _):
            for u in range(4):
                k = k4 * 4 + u
                rb = u % NROWS
                rb1 = (u + 1) % NROWS
                bb = u % NBLOB
                bb1 = (u + 1) % NBLOB
                bb2 = (u + 2) % NBLOB

                wait_gather(rb)

                @pl.when(k + 1 < KPT)
                def _():
                    wait_blob(bb1)

                    @pl.when(k >= 1)
                    def _():
                        wait_scatter(rb1, 0)
                        wait_scatter(rb1, 1)
                    fire_gather(k + 1, bb1, rb1)

                for half in range(2):
                    for j in range(H // LANES):
                        didx[rb][half][pl.ds(j * LANES, LANES)] = (
                            blob[bb][1, pl.ds(half * H + j * LANES, LANES)])

                def scale_row(i, _):
                    wb = plsc.bitcast(
                        plsc.load_gather(
                            blob[bb],
                            [jnp.full((LANES,), 2, jnp.int32),
                             jnp.full((LANES,), 0, jnp.int32) + i]),
                        jnp.float32)
                    for j in range(D // LANES):
                        sl = pl.ds(j * LANES, LANES)
                        rows[rb][i, sl] = rows[rb][i, sl] * wb
                    return 0
                lax.fori_loop(0, H, scale_row, 0, unroll=4)
                fire_scatter(rb, 0)
                lax.fori_loop(H, C, scale_row, 0, unroll=4)
                fire_scatter(rb, 1)

                @pl.when(k + 2 < KPT)
                def _():
                    fire_blob(k + 2, bb2)
            return 0
        lax.fori_loop(0, KPT // 4, outer, 0)

        for kk in (KPT - 2, KPT - 1):
            wait_scatter(kk % NROWS, 0)
            wait_scatter(kk % NROWS, 1)
        plsc.subcore_barrier()

        for t in range(rows_per_tile // zchunk):
            sl = pl.ds(s * rows_per_tile + t * zchunk, zchunk)
            pltpu.sync_copy(acc.at[sl], out_hbm.at[c, sl])

    return agg_kernel


_BLK = 1024


def _dis_from_degp(degp_blk):
    deg = degp_blk[0, :] + degp_blk[1, :] + 1.0
    return lax.rsqrt(deg)


def _layer_norm(z, g, b, eps=1e-5):
    mu = jnp.mean(z, axis=-1, keepdims=True)
    var = jnp.mean((z - mu) ** 2, axis=-1, keepdims=True)
    return (z - mu) * lax.rsqrt(var + eps) * g + b


def _tc1_body(degp_ref, x_ref, w_ref, hp_ref):
    dis = _dis_from_degp(degp_ref[...])
    h = jnp.dot(x_ref[...], w_ref[...], preferred_element_type=jnp.float32,
                precision=lax.Precision.HIGHEST)
    hp_ref[...] = h * dis[:, None]


def _tc2_body(p_ref, hp_ref, degp_ref, w2_ref, b1_ref, g1_ref, be1_ref, out_ref):
    dis = _dis_from_degp(degp_ref[...])
    z = (p_ref[0] + p_ref[1] + hp_ref[...]) * dis[:, None] + b1_ref[...]
    z = _layer_norm(z, g1_ref[...], be1_ref[...])
    r = jnp.maximum(z, 0.0)
    h2 = jnp.dot(r, w2_ref[...], preferred_element_type=jnp.float32,
                 precision=lax.Precision.HIGHEST)
    out_ref[...] = h2 * dis[:, None]


def _tc3_body(p_ref, hp_ref, degp_ref, b2_ref, g2_ref, be2_ref, out_ref):
    dis = _dis_from_degp(degp_ref[...])
    z = (p_ref[0] + p_ref[1] + hp_ref[...]) * dis[:, None] + b2_ref[...]
    out_ref[...] = _layer_norm(z, g2_ref[...], be2_ref[...])


def _row_spec(D):
    return pl.BlockSpec((_BLK, D), lambda i: (i, 0))


def _p_spec(D):
    return pl.BlockSpec((2, _BLK, D), lambda i: (0, i, 0))


def _full_spec(shape):
    nd = len(shape)
    return pl.BlockSpec(shape, lambda i: (0,) * nd)


def kernel(x, edge_index, edge_weight, W1, b1, g1, be1, W2, b2, g2, be2):
    N, D = x.shape
    E = edge_weight.shape[0]
    NPAD = ((N + NW * LANES - 1) // (NW * LANES)) * (NW * LANES)

    E_pad = NW * KPT * C
    pad = E_pad - E
    pad_idx = (jnp.arange(pad, dtype=jnp.int32) % N)
    src = jnp.concatenate([edge_index[0].astype(jnp.int32), pad_idx]).reshape(-1, C)
    dst = jnp.concatenate([edge_index[1].astype(jnp.int32), pad_idx]).reshape(-1, C)
    w = jnp.concatenate([edge_weight.astype(jnp.float32),
                         jnp.zeros((pad,), jnp.float32)]).reshape(-1, C)
    wbits = lax.bitcast_convert_type(w, jnp.int32)
    blob = jnp.stack([src, dst, wbits], axis=1)

    degp = _make_deg_kernel(NPAD)(dst, w)
    deg_t = degp[:, :N]

    grid = (N + _BLK - 1) // _BLK
    degp_spec = pl.BlockSpec((2, _BLK), lambda i: (0, i))

    hp1 = pl.pallas_call(
        _tc1_body,
        grid=(grid,),
        in_specs=[degp_spec, _row_spec(D), _full_spec((D, D))],
        out_specs=_row_spec(D),
        out_shape=jax.ShapeDtypeStruct((N, D), jnp.float32),
    )(deg_t, x, W1)

    agg = _make_agg_kernel(NPAD, D)

    P1 = agg(hp1, blob)

    hp2 = pl.pallas_call(
        _tc2_body,
        grid=(grid,),
        in_specs=[_p_spec(D), _row_spec(D), degp_spec, _full_spec((D, D)),
                  _full_spec((1, D)), _full_spec((1, D)), _full_spec((1, D))],
        out_specs=_row_spec(D),
        out_shape=jax.ShapeDtypeStruct((N, D), jnp.float32),
    )(P1, hp1, deg_t, W2, b1.reshape(1, D), g1.reshape(1, D), be1.reshape(1, D))

    P2 = agg(hp2, blob)

    out = pl.pallas_call(
        _tc3_body,
        grid=(grid,),
        in_specs=[_p_spec(D), _row_spec(D), degp_spec,
                  _full_spec((1, D)), _full_spec((1, D)), _full_spec((1, D))],
        out_specs=_row_spec(D),
        out_shape=jax.ShapeDtypeStruct((N, D), jnp.float32),
    )(P2, hp2, deg_t, b2.reshape(1, D), g2.reshape(1, D), be2.reshape(1, D))

    return out


# speedup vs baseline: 1.5704x; 1.1872x over previous
import functools

import jax
import jax.numpy as jnp
from jax import lax
from jax.experimental import pallas as pl
from jax.experimental.pallas import tpu as pltpu
from jax.experimental.pallas import tpu_sc as plsc

NC = 2
NS = 16
NW = NC * NS
LANES = 16
C = 128
KPT = 80
NROWS = 2
NBLOB = 4


def _make_deg_kernel(NPAD):
    mesh = plsc.VectorSubcoreMesh(core_axis_name="c", subcore_axis_name="s",
                                  num_cores=NC, num_subcores=NS)
    rows_per_tile = NPAD // NS

    @functools.partial(
        pl.kernel, mesh=mesh,
        out_type=jax.ShapeDtypeStruct((NC, NPAD), jnp.float32),
        compiler_params=pltpu.CompilerParams(needs_layout_passes=False),
        scratch_types=[
            pltpu.VMEM((KPT, C), jnp.int32),
            pltpu.VMEM((KPT, C), jnp.float32),
            pltpu.VMEM((rows_per_tile,), jnp.float32),
            pltpu.VMEM_SHARED((NPAD,), jnp.float32),
        ],
    )
    def deg_kernel(dst_hbm, w_hbm, out_hbm, didx, w_v, zbuf, deg_sh):
        c = lax.axis_index("c")
        s = lax.axis_index("s")
        wid = c * NS + s

        def zero_body(k, _):
            zbuf[pl.ds(k * LANES, LANES)] = jnp.zeros((LANES,), jnp.float32)
            return 0
        lax.fori_loop(0, rows_per_tile // LANES, zero_body, 0)
        pltpu.sync_copy(zbuf, deg_sh.at[pl.ds(s * rows_per_tile, rows_per_tile)])
        pltpu.sync_copy(dst_hbm.at[pl.ds(wid * KPT, KPT)], didx)
        pltpu.sync_copy(w_hbm.at[pl.ds(wid * KPT, KPT)], w_v)
        plsc.subcore_barrier()

        def chunk_body(k, _):
            pltpu.sync_copy(w_v.at[k], deg_sh.at[didx.at[k]], add=True)
            return 0
        lax.fori_loop(0, KPT, chunk_body, 0)
        plsc.subcore_barrier()

        sl = pl.ds(s * rows_per_tile, rows_per_tile)
        pltpu.sync_copy(deg_sh.at[sl], out_hbm.at[c, sl])

    return deg_kernel


def _make_agg_kernel(NPAD, D):
    mesh = plsc.VectorSubcoreMesh(core_axis_name="c", subcore_axis_name="s",
                                  num_cores=NC, num_subcores=NS)
    rows_per_tile = NPAD // NS
    HC = C // 2
    NU = 2 * KPT

    @functools.partial(
        pl.kernel, mesh=mesh,
        out_type=jax.ShapeDtypeStruct((NC, NPAD, D), jnp.float32),
        compiler_params=pltpu.CompilerParams(needs_layout_passes=False),
        scratch_types=[
            [pltpu.VMEM((3, C), jnp.int32) for _ in range(NBLOB)],
            [pltpu.VMEM((HC, D), jnp.float32) for _ in range(4)],
            [pltpu.VMEM((HC,), jnp.int32) for _ in range(4)],
            pltpu.VMEM_SHARED((NPAD, D), jnp.float32),
            [pltpu.SemaphoreType.DMA for _ in range(NBLOB)],
            [pltpu.SemaphoreType.DMA for _ in range(4)],
            [pltpu.SemaphoreType.DMA for _ in range(4)],
        ],
    )
    def agg_kernel(hp_hbm, blob_hbm, out_hbm, blob, rows, didx, acc,
                   bsem, gsem, ssem):
        c = lax.axis_index("c")
        s = lax.axis_index("s")
        wid = c * NS + s
        base = wid * KPT

        def zero_row(i, _):
            for j in range(D // LANES):
                rows[0][i, pl.ds(j * LANES, LANES)] = jnp.zeros((LANES,), jnp.float32)
            return 0
        lax.fori_loop(0, HC, zero_row, 0)
        for t in range(rows_per_tile // HC):
            pltpu.sync_copy(rows[0],
                            acc.at[pl.ds(s * rows_per_tile + t * HC, HC)])
        plsc.subcore_barrier()

        def fire_blob(k, bb):
            pltpu.async_copy(blob_hbm.at[base + k], blob[bb], bsem[bb])

        def wait_blob(bb):
            pltpu.make_async_copy(blob_hbm.at[0], blob[bb], bsem[bb]).wait()

        def fire_gather(k, half, bb, rb):
            pltpu.async_copy(hp_hbm.at[blob[bb].at[0, pl.ds(half * HC, HC)]],
                             rows[rb], gsem[rb])

        def wait_gather(rb):
            pltpu.make_async_copy(hp_hbm.at[pl.ds(0, HC)], rows[rb],
                                  gsem[rb]).wait()

        def fire_scatter(rb):
            pltpu.async_copy(rows[rb], acc.at[didx[rb]], ssem[rb], add=True)

        def wait_scatter(rb):
            pltpu.make_async_copy(hp_hbm.at[pl.ds(0, HC)], rows[rb],
                                  ssem[rb]).wait()

        fire_blob(0, 0)
        fire_blob(1, 1)
        wait_blob(0)
        fire_gather(0, 0, 0, 0)
        fire_gather(0, 1, 0, 1)

        def outer(j8, _):
            for u in range(8):
                b = u % 4
                half = u % 2
                kb = (u // 2) % NBLOB
                k = j8 * 4 + (u // 2)
                j = j8 * 8 + u
                b2 = (u + 2) % 4
                kb2 = ((u + 2) // 2) % NBLOB

                @pl.when(j + 2 < NU)
                def _():
                    if (u + 2) % 2 == 0:
                        wait_blob(kb2)

                    @pl.when(j >= 2)
                    def _():
                        wait_scatter(b2)
                    fire_gather(k + 1, half, kb2, b2)

                wait_gather(b)

                for t in range(HC // LANES):
                    didx[b][pl.ds(t * LANES, LANES)] = (
                        blob[kb][1, pl.ds(half * HC + t * LANES, LANES)])

                def scale_row(i, _):
                    wb = plsc.bitcast(
                        plsc.load_gather(
                            blob[kb],
                            [jnp.full((LANES,), 2, jnp.int32),
                             jnp.full((LANES,), half * HC, jnp.int32) + i]),
                        jnp.float32)
                    for jj in range(D // LANES):
                        sl = pl.ds(jj * LANES, LANES)
                        rows[b][i, sl] = rows[b][i, sl] * wb
                    return 0
                lax.fori_loop(0, HC, scale_row, 0, unroll=4)

                fire_scatter(b)

                if u % 2 == 0:
                    kbn = ((u // 2) + 2) % NBLOB

                    @pl.when(k + 2 < KPT)
                    def _():
                        fire_blob(k + 2, kbn)
            return 0
        lax.fori_loop(0, NU // 8, outer, 0)

        for b in range(4):
            wait_scatter(b)
        plsc.subcore_barrier()

        for t in range(rows_per_tile // HC):
            sl = pl.ds(s * rows_per_tile + t * HC, HC)
            pltpu.sync_copy(acc.at[sl], out_hbm.at[c, sl])

    return agg_kernel


_BLK = 1024


def _dis_from_degp(degp_blk):
    deg = degp_blk[0, :] + degp_blk[1, :] + 1.0
    return lax.rsqrt(deg)


def _layer_norm(z, g, b, eps=1e-5):
    mu = jnp.mean(z, axis=-1, keepdims=True)
    var = jnp.mean((z - mu) ** 2, axis=-1, keepdims=True)
    return (z - mu) * lax.rsqrt(var + eps) * g + b


def _tc1_body(degp_ref, x_ref, w_ref, hp_ref):
    dis = _dis_from_degp(degp_ref[...])
    h = jnp.dot(x_ref[...], w_ref[...], preferred_element_type=jnp.float32,
                precision=lax.Precision.HIGHEST)
    hp_ref[...] = h * dis[:, None]


def _tc2_body(p_ref, hp_ref, degp_ref, w2_ref, b1_ref, g1_ref, be1_ref, out_ref):
    dis = _dis_from_degp(degp_ref[...])
    z = (p_ref[0] + p_ref[1] + hp_ref[...]) * dis[:, None] + b1_ref[...]
    z = _layer_norm(z, g1_ref[...], be1_ref[...])
    r = jnp.maximum(z, 0.0)
    h2 = jnp.dot(r, w2_ref[...], preferred_element_type=jnp.float32,
                 precision=lax.Precision.HIGHEST)
    out_ref[...] = h2 * dis[:, None]


def _tc3_body(p_ref, hp_ref, degp_ref, b2_ref, g2_ref, be2_ref, out_ref):
    dis = _dis_from_degp(degp_ref[...])
    z = (p_ref[0] + p_ref[1] + hp_ref[...]) * dis[:, None] + b2_ref[...]
    out_ref[...] = _layer_norm(z, g2_ref[...], be2_ref[...])


def _row_spec(D):
    return pl.BlockSpec((_BLK, D), lambda i: (i, 0))


def _p_spec(D):
    return pl.BlockSpec((2, _BLK, D), lambda i: (0, i, 0))


def _full_spec(shape):
    nd = len(shape)
    return pl.BlockSpec(shape, lambda i: (0,) * nd)


def kernel(x, edge_index, edge_weight, W1, b1, g1, be1, W2, b2, g2, be2):
    N, D = x.shape
    E = edge_weight.shape[0]
    NPAD = ((N + NW * LANES - 1) // (NW * LANES)) * (NW * LANES)

    E_pad = NW * KPT * C
    pad = E_pad - E
    pad_idx = (jnp.arange(pad, dtype=jnp.int32) % N)
    src = jnp.concatenate([edge_index[0].astype(jnp.int32), pad_idx]).reshape(-1, C)
    dst = jnp.concatenate([edge_index[1].astype(jnp.int32), pad_idx]).reshape(-1, C)
    w = jnp.concatenate([edge_weight.astype(jnp.float32),
                         jnp.zeros((pad,), jnp.float32)]).reshape(-1, C)
    wbits = lax.bitcast_convert_type(w, jnp.int32)
    blob = jnp.stack([src, dst, wbits], axis=1)

    degp = _make_deg_kernel(NPAD)(dst, w)
    deg_t = degp[:, :N]

    grid = (N + _BLK - 1) // _BLK
    degp_spec = pl.BlockSpec((2, _BLK), lambda i: (0, i))

    hp1 = pl.pallas_call(
        _tc1_body,
        grid=(grid,),
        in_specs=[degp_spec, _row_spec(D), _full_spec((D, D))],
        out_specs=_row_spec(D),
        out_shape=jax.ShapeDtypeStruct((N, D), jnp.float32),
    )(deg_t, x, W1)

    agg = _make_agg_kernel(NPAD, D)

    P1 = agg(hp1, blob)

    hp2 = pl.pallas_call(
        _tc2_body,
        grid=(grid,),
        in_specs=[_p_spec(D), _row_spec(D), degp_spec, _full_spec((D, D)),
                  _full_spec((1, D)), _full_spec((1, D)), _full_spec((1, D))],
        out_specs=_row_spec(D),
        out_shape=jax.ShapeDtypeStruct((N, D), jnp.float32),
    )(P1, hp1, deg_t, W2, b1.reshape(1, D), g1.reshape(1, D), be1.reshape(1, D))

    P2 = agg(hp2, blob)

    out = pl.pallas_call(
        _tc3_body,
        grid=(grid,),
        in_specs=[_p_spec(D), _row_spec(D), degp_spec,
                  _full_spec((1, D)), _full_spec((1, D)), _full_spec((1, D))],
        out_specs=_row_spec(D),
        out_shape=jax.ShapeDtypeStruct((N, D), jnp.float32),
    )(P2, hp2, deg_t, b2.reshape(1, D), g2.reshape(1, D), be2.reshape(1, D))

    return out
